# baseline (device time: 19998 ns/iter reference)
import jax
import jax.numpy as jnp
from jax import lax
from jax.experimental import pallas as pl
from jax.experimental.pallas import tpu as pltpu

N_DEV = 4


def kernel(x):
    x = x.astype(jnp.bfloat16)
    m, n_total = x.shape
    blk = n_total // N_DEV
    out_rows = N_DEV * m

    def body(x_ref, out_ref, send_sems, recv_sems, local_sem):
        me = lax.axis_index("i")

        barrier_sem = pltpu.get_barrier_semaphore()
        for o in range(1, N_DEV):
            pl.semaphore_signal(
                barrier_sem,
                inc=1,
                device_id=((me + o) % N_DEV,),
                device_id_type=pl.DeviceIdType.MESH,
            )
        pl.semaphore_wait(barrier_sem, N_DEV - 1)

        sends = []
        for o in (2, 1, 3):
            t = (me + o) % N_DEV
            rdma = pltpu.make_async_remote_copy(
                src_ref=x_ref.at[:, pl.ds(t * blk, blk)],
                dst_ref=out_ref.at[pl.ds(me * m, m), :],
                send_sem=send_sems.at[o],
                recv_sem=recv_sems.at[o],
                device_id=(t,),
                device_id_type=pl.DeviceIdType.MESH,
            )
            rdma.start()
            sends.append(rdma)

        local = pltpu.make_async_copy(
            x_ref.at[:, pl.ds(me * blk, blk)],
            out_ref.at[pl.ds(me * m, m), :],
            local_sem,
        )
        local.start()

        for o in range(1, N_DEV):
            s = (me - o) % N_DEV
            recv = pltpu.make_async_remote_copy(
                src_ref=x_ref.at[:, pl.ds(s * blk, blk)],
                dst_ref=out_ref.at[pl.ds(s * m, m), :],
                send_sem=send_sems.at[o],
                recv_sem=recv_sems.at[o],
                device_id=(s,),
                device_id_type=pl.DeviceIdType.MESH,
            )
            recv.wait_recv()

        local.wait()
        for rdma in sends:
            rdma.wait_send()

    return pl.pallas_call(
        body,
        out_shape=jax.ShapeDtypeStruct((out_rows, blk), jnp.bfloat16),
        in_specs=[pl.BlockSpec(memory_space=pltpu.VMEM)],
        out_specs=pl.BlockSpec(memory_space=pl.ANY),
        scratch_shapes=[
            pltpu.SemaphoreType.DMA((N_DEV,)),
            pltpu.SemaphoreType.DMA((N_DEV,)),
            pltpu.SemaphoreType.DMA,
        ],
        compiler_params=pltpu.CompilerParams(collective_id=0),
    )(x)


# device time: 19756 ns/iter; 1.0122x vs baseline; 1.0122x over previous
import jax
import jax.numpy as jnp
from jax import lax
from jax.experimental import pallas as pl
from jax.experimental.pallas import tpu as pltpu

N_DEV = 4


def kernel(x):
    m, n_total = x.shape
    blk = n_total // N_DEV
    out_rows = N_DEV * m

    order = (2, 1, 3)

    def body(x_ref, out_ref, send_buf, send_sems, recv_sems, local_sem):
        me = lax.axis_index("i")

        def cast_block(t):
            send_buf[:, pl.ds(t * blk, blk)] = x_ref[
                :, pl.ds(t * blk, blk)
            ].astype(jnp.bfloat16)

        cast_block((me + order[0]) % N_DEV)

        barrier_sem = pltpu.get_barrier_semaphore()
        for o in range(1, N_DEV):
            pl.semaphore_signal(
                barrier_sem,
                inc=1,
                device_id=((me + o) % N_DEV,),
                device_id_type=pl.DeviceIdType.MESH,
            )
        pl.semaphore_wait(barrier_sem, N_DEV - 1)

        sends = []
        for k, o in enumerate(order):
            t = (me + o) % N_DEV
            rdma = pltpu.make_async_remote_copy(
                src_ref=send_buf.at[:, pl.ds(t * blk, blk)],
                dst_ref=out_ref.at[pl.ds(me * m, m), :],
                send_sem=send_sems.at[o],
                recv_sem=recv_sems.at[o],
                device_id=(t,),
                device_id_type=pl.DeviceIdType.MESH,
            )
            rdma.start()
            sends.append(rdma)
            if k + 1 < len(order):
                cast_block((me + order[k + 1]) % N_DEV)

        cast_block(me)
        local = pltpu.make_async_copy(
            send_buf.at[:, pl.ds(me * blk, blk)],
            out_ref.at[pl.ds(me * m, m), :],
            local_sem,
        )
        local.start()

        for o in range(1, N_DEV):
            s = (me - o) % N_DEV
            recv = pltpu.make_async_remote_copy(
                src_ref=send_buf.at[:, pl.ds(s * blk, blk)],
                dst_ref=out_ref.at[pl.ds(s * m, m), :],
                send_sem=send_sems.at[o],
                recv_sem=recv_sems.at[o],
                device_id=(s,),
                device_id_type=pl.DeviceIdType.MESH,
            )
            recv.wait_recv()

        local.wait()
        for rdma in sends:
            rdma.wait_send()

    return pl.pallas_call(
        body,
        out_shape=jax.ShapeDtypeStruct((out_rows, blk), jnp.bfloat16),
        in_specs=[pl.BlockSpec(memory_space=pltpu.VMEM)],
        out_specs=pl.BlockSpec(memory_space=pl.ANY),
        scratch_shapes=[
            pltpu.VMEM((m, n_total), jnp.bfloat16),
            pltpu.SemaphoreType.DMA((N_DEV,)),
            pltpu.SemaphoreType.DMA((N_DEV,)),
            pltpu.SemaphoreType.DMA,
        ],
        compiler_params=pltpu.CompilerParams(collective_id=0),
    )(x)
